# baseline (speedup 1.0000x reference)
import functools

import jax
import jax.numpy as jnp
from jax.experimental import pallas as pl
from jax.experimental.pallas import tpu as pltpu

_B, _C, _H, _W = 8, 96, 56, 56
_HW = _H * _W
_HID = _C // 16
_BC = _B * _C
_N = 32
_G = 48


def _perm_const():
    pkey = jax.random.key(42)
    return jax.random.permutation(jax.random.fold_in(pkey, 0), _C)


def _fused_body(x_ref, w1_ref, b1_ref, w2_ref, b2_ref, perm_ref, o_ref,
                xbuf, obuf, sg_vmem, sg_smem, in_sem, sg_sem, out_sems):
    pltpu.make_async_copy(x_ref, xbuf, in_sem).start()
    pltpu.make_async_copy(x_ref, xbuf, in_sem).wait()

    s = jnp.mean(xbuf[:, :, 0], axis=(2, 3))
    h = jnp.maximum(
        jax.lax.dot_general(s, w1_ref[...], (((1,), (1,)), ((), ())))
        + b1_ref[...], 0.0)
    lg = jax.lax.dot_general(h, w2_ref[...], (((1,), (1,)), ((), ())))
    sc = jax.nn.sigmoid(lg + b2_ref[...])

    gt = (sc[:, None, :] > sc[:, :, None])
    eq = (sc[:, None, :] == sc[:, :, None])
    ii = jax.lax.broadcasted_iota(jnp.int32, (_B, _C, _C), 1)
    jj = jax.lax.broadcasted_iota(jnp.int32, (_B, _C, _C), 2)
    r = jnp.sum((gt | (eq & (jj < ii))).astype(jnp.int32), axis=2)
    match = (r[:, :, None] == perm_ref[...][0][None, None, :])
    sg = jnp.sum(jnp.where(match, ii, 0), axis=1)

    sg_vmem[...] = sg
    pltpu.make_async_copy(sg_vmem, sg_smem, sg_sem).start()
    pltpu.make_async_copy(sg_vmem, sg_smem, sg_sem).wait()

    def out_copy(t, slot):
        b, g = t // 2, t % 2
        return pltpu.make_async_copy(
            obuf.at[slot],
            o_ref.at[:, pl.ds(b, 1), pl.ds(g * _G, _G)],
            out_sems.at[slot])

    for t in range(2 * _B):
        b, g = t // 2, t % 2
        slot = t % 2
        if t >= 2:
            out_copy(t - 2, slot).wait()
        for jj_ in range(_G):
            j = g * _G + jj_
            c = sg_smem[b, j]
            obuf[slot, :, 0, jj_] = xbuf[:, pl.ds(c, 1), 0, :, :][:, 0]
        out_copy(t, slot).start()

    out_copy(2 * _B - 2, 0).wait()
    out_copy(2 * _B - 1, 1).wait()


@jax.jit
def kernel(x, W1, b1, W2, b2):
    perm = _perm_const().astype(jnp.int32).reshape(1, _C)

    out = pl.pallas_call(
        _fused_body,
        in_specs=[
            pl.BlockSpec(memory_space=pltpu.MemorySpace.HBM),
            pl.BlockSpec(memory_space=pltpu.MemorySpace.VMEM),
            pl.BlockSpec(memory_space=pltpu.MemorySpace.VMEM),
            pl.BlockSpec(memory_space=pltpu.MemorySpace.VMEM),
            pl.BlockSpec(memory_space=pltpu.MemorySpace.VMEM),
            pl.BlockSpec(memory_space=pltpu.MemorySpace.VMEM),
        ],
        out_specs=pl.BlockSpec(memory_space=pltpu.MemorySpace.HBM),
        out_shape=jax.ShapeDtypeStruct((_B, _B, _C, _H, _W), jnp.float32),
        scratch_shapes=[
            pltpu.VMEM((_B, _C, 1, _H, _W), jnp.float32),
            pltpu.VMEM((2, _B, 1, _G, _H, _W), jnp.float32),
            pltpu.VMEM((_B, _C), jnp.int32),
            pltpu.SMEM((_B, _C), jnp.int32),
            pltpu.SemaphoreType.DMA,
            pltpu.SemaphoreType.DMA,
            pltpu.SemaphoreType.DMA((2,)),
        ],
    )(x[:, :, None], W1, b1.reshape(1, _HID), W2, b2.reshape(1, _C), perm)

    return out
